# baseline (speedup 1.0000x reference)
import jax
import jax.numpy as jnp
from jax import lax
from jax.experimental import pallas as pl
from jax.experimental.pallas import tpu as pltpu
from jax.experimental.pallas import tpu_sc as plsc

B, C, H, W = 8, 384, 56, 56
P1 = C // 2
N = B * H * W
NW = 32
RPW = N // NW
R = 16
NCH = RPW // R
NK = P1 // 16
RB = 512
GRID = N // RB


def _sc_body(x0, x1, bn1, thrh, o0,
             bn1_v, thr_v, a0, a1, b0, b1, c0, c1, sem_in, sem_out):
    wid = lax.axis_index("s") * 2 + lax.axis_index("c")
    base = wid * RPW

    pltpu.sync_copy(bn1, bn1_v)
    pltpu.sync_copy(thrh, thr_v)
    thr = thr_v[...]

    def fire_in(i, bx0, bx1):
        r0 = base + i * R
        pltpu.async_copy(x0.at[pl.ds(r0, R), :], bx0, sem_in)
        pltpu.async_copy(x1.at[pl.ds(r0, R), :], bx1, sem_in)

    def wait_in():
        pltpu.make_async_copy(x0.at[pl.ds(0, R), :], a0, sem_in).wait()
        pltpu.make_async_copy(x1.at[pl.ds(0, R), :], a1, sem_in).wait()

    def fire_out(i, bx0):
        r0 = base + i * R
        pltpu.async_copy(bx0, o0.at[pl.ds(r0, R), :], sem_out)

    def wait_out():
        pltpu.make_async_copy(a0, o0.at[pl.ds(0, R), :], sem_out).wait()

    def compute(bx0, bx1):
        zero = jnp.zeros((16,), jnp.float32)
        for k in range(NK):
            cs = P1 + k * 16
            q1 = jnp.abs(bn1_v[pl.ds(cs, 16)])
            m0a = q1 > thr
            m0b = q1 < thr
            for r in range(R):
                x0c = bx0[r, pl.ds(cs, 16)]
                x1c = bx1[r, pl.ds(cs, 16)]
                bx0[r, pl.ds(cs, 16)] = jnp.where(
                    m0a, x0c, jnp.where(m0b, x1c, zero))

    def half(i, bx0, bx1, nx0, nx1):
        wait_in()
        compute(bx0, bx1)
        fire_out(i, bx0)

        @pl.when(i + 2 < NCH)
        def _():
            @pl.when(i >= 1)
            def _():
                wait_out()
            fire_in(i + 2, nx0, nx1)

    fire_in(0, a0, a1)
    fire_in(1, b0, b1)

    def body(p, carry):
        i0 = 3 * p
        half(i0, a0, a1, c0, c1)
        half(i0 + 1, b0, b1, a0, a1)
        half(i0 + 2, c0, c1, b0, b1)
        return carry

    lax.fori_loop(0, NCH // 3, body, 0)
    wait_in()
    compute(a0, a1)
    fire_out(NCH - 1, a0)
    wait_out()
    wait_out()
    wait_out()


def _tc_body(thr_ref, bn2_ref, x0_ref, x1_ref, o1_ref):
    thr = thr_ref[0, 0]
    c_idx = jax.lax.broadcasted_iota(jnp.int32, (1, C), 1)
    first = c_idx < P1
    bn2 = jnp.abs(bn2_ref[...])
    keep1 = jnp.logical_or(first, bn2 > thr)
    take1 = jnp.logical_and(jnp.logical_not(first), bn2 < thr)
    x0 = x0_ref[...]
    x1 = x1_ref[...]
    zero = jnp.zeros_like(x0)
    o1_ref[...] = jnp.where(keep1, x1, jnp.where(take1, x0, zero))


@jax.jit
def _run(x0, x1, bn1, bn2, thr):
    x0r = x0.transpose(0, 2, 3, 1).reshape(N, C)
    x1r = x1.transpose(0, 2, 3, 1).reshape(N, C)
    thrh = jnp.full((16,), thr, jnp.float32)
    mesh = plsc.VectorSubcoreMesh(core_axis_name="c", subcore_axis_name="s")
    sc = pl.kernel(
        _sc_body,
        out_type=jax.ShapeDtypeStruct((N, C), jnp.float32),
        mesh=mesh,
        scratch_types=[
            pltpu.VMEM((C,), jnp.float32),
            pltpu.VMEM((16,), jnp.float32),
            pltpu.VMEM((R, C), jnp.float32),
            pltpu.VMEM((R, C), jnp.float32),
            pltpu.VMEM((R, C), jnp.float32),
            pltpu.VMEM((R, C), jnp.float32),
            pltpu.VMEM((R, C), jnp.float32),
            pltpu.VMEM((R, C), jnp.float32),
            pltpu.SemaphoreType.DMA,
            pltpu.SemaphoreType.DMA,
        ],
        compiler_params=pltpu.CompilerParams(use_tc_tiling_on_sc=True),
    )
    o0 = sc(x0r, x1r, bn1, thrh)

    bn2r = bn2.reshape(1, C)
    thr_arr = jnp.asarray(thr, jnp.float32).reshape(1, 1)
    data_spec = pl.BlockSpec((RB, C), lambda i: (i, 0))
    vec_spec = pl.BlockSpec((1, C), lambda i: (0, 0))
    thr_spec = pl.BlockSpec((1, 1), lambda i: (0, 0))
    o1 = pl.pallas_call(
        _tc_body,
        grid=(GRID,),
        in_specs=[thr_spec, vec_spec, data_spec, data_spec],
        out_specs=data_spec,
        out_shape=jax.ShapeDtypeStruct((N, C), jnp.float32),
        compiler_params=pltpu.CompilerParams(
            dimension_semantics=("parallel",),
        ),
    )(thr_arr, bn2r, x0r, x1r)

    o0 = o0.reshape(B, H, W, C).transpose(0, 3, 1, 2)
    o1 = o1.reshape(B, H, W, C).transpose(0, 3, 1, 2)
    return o0, o1


def kernel(x0, x1, bn1_weight, bn2_weight, bn_threshold):
    return _run(x0, x1, bn1_weight, bn2_weight, bn_threshold)


# speedup vs baseline: 1.8191x; 1.8191x over previous
import jax
import jax.numpy as jnp
from jax.experimental import pallas as pl
from jax.experimental.pallas import tpu as pltpu

B, C, H, W = 8, 384, 56, 56
P1 = C // 2
N = B * H * W
RB = 896
GRID = N // RB


def _body(thr_ref, bn1_ref, bn2_ref, x0_ref, x1_ref, o0_ref, o1_ref):
    thr = thr_ref[0, 0]
    c_idx = jax.lax.broadcasted_iota(jnp.int32, (1, C), 1)
    first = c_idx < P1
    bn1 = jnp.abs(bn1_ref[...])
    bn2 = jnp.abs(bn2_ref[...])
    keep0 = jnp.logical_or(first, bn1 > thr)
    take0 = jnp.logical_and(jnp.logical_not(first), bn1 < thr)
    keep1 = jnp.logical_or(first, bn2 > thr)
    take1 = jnp.logical_and(jnp.logical_not(first), bn2 < thr)
    x0 = x0_ref[...]
    x1 = x1_ref[...]
    zero = jnp.zeros_like(x0)
    o0_ref[...] = jnp.where(keep0, x0, jnp.where(take0, x1, zero))
    o1_ref[...] = jnp.where(keep1, x1, jnp.where(take1, x0, zero))


@jax.jit
def _run(x0, x1, bn1, bn2, thr):
    x0r = x0.transpose(0, 2, 3, 1).reshape(N, C)
    x1r = x1.transpose(0, 2, 3, 1).reshape(N, C)
    bn1r = bn1.reshape(1, C)
    bn2r = bn2.reshape(1, C)
    thr_arr = jnp.asarray(thr, jnp.float32).reshape(1, 1)
    data_spec = pl.BlockSpec((RB, C), lambda i: (i, 0))
    vec_spec = pl.BlockSpec((1, C), lambda i: (0, 0))
    thr_spec = pl.BlockSpec((1, 1), lambda i: (0, 0))
    o0, o1 = pl.pallas_call(
        _body,
        grid=(GRID,),
        in_specs=[thr_spec, vec_spec, vec_spec, data_spec, data_spec],
        out_specs=[data_spec, data_spec],
        out_shape=[
            jax.ShapeDtypeStruct((N, C), jnp.float32),
            jax.ShapeDtypeStruct((N, C), jnp.float32),
        ],
        compiler_params=pltpu.CompilerParams(
            dimension_semantics=("parallel",),
        ),
    )(thr_arr, bn1r, bn2r, x0r, x1r)
    o0 = o0.reshape(B, H, W, C).transpose(0, 3, 1, 2)
    o1 = o1.reshape(B, H, W, C).transpose(0, 3, 1, 2)
    return o0, o1


def kernel(x0, x1, bn1_weight, bn2_weight, bn_threshold):
    return _run(x0, x1, bn1_weight, bn2_weight, bn_threshold)


# speedup vs baseline: 1.9407x; 1.0668x over previous
import jax
import jax.numpy as jnp
from jax.experimental import pallas as pl
from jax.experimental.pallas import tpu as pltpu

B, C, H, W = 8, 384, 56, 56
P1 = C // 2
N = B * H * W
RB = 1792
GRID = N // RB


def _body(thr_ref, bn1_ref, bn2_ref, x0_ref, x1_ref, o0_ref, o1_ref):
    thr = thr_ref[0, 0]
    c_idx = jax.lax.broadcasted_iota(jnp.int32, (1, C), 1)
    first = c_idx < P1
    bn1 = jnp.abs(bn1_ref[...])
    bn2 = jnp.abs(bn2_ref[...])
    keep0 = jnp.logical_or(first, bn1 > thr)
    take0 = jnp.logical_and(jnp.logical_not(first), bn1 < thr)
    keep1 = jnp.logical_or(first, bn2 > thr)
    take1 = jnp.logical_and(jnp.logical_not(first), bn2 < thr)
    x0 = x0_ref[...]
    x1 = x1_ref[...]
    zero = jnp.zeros_like(x0)
    o0_ref[...] = jnp.where(keep0, x0, jnp.where(take0, x1, zero))
    o1_ref[...] = jnp.where(keep1, x1, jnp.where(take1, x0, zero))


@jax.jit
def _run(x0, x1, bn1, bn2, thr):
    x0r = x0.transpose(0, 2, 3, 1).reshape(N, C)
    x1r = x1.transpose(0, 2, 3, 1).reshape(N, C)
    bn1r = bn1.reshape(1, C)
    bn2r = bn2.reshape(1, C)
    thr_arr = jnp.asarray(thr, jnp.float32).reshape(1, 1)
    data_spec = pl.BlockSpec((RB, C), lambda i: (i, 0))
    vec_spec = pl.BlockSpec((1, C), lambda i: (0, 0))
    thr_spec = pl.BlockSpec((1, 1), lambda i: (0, 0))
    o0, o1 = pl.pallas_call(
        _body,
        grid=(GRID,),
        in_specs=[thr_spec, vec_spec, vec_spec, data_spec, data_spec],
        out_specs=[data_spec, data_spec],
        out_shape=[
            jax.ShapeDtypeStruct((N, C), jnp.float32),
            jax.ShapeDtypeStruct((N, C), jnp.float32),
        ],
        compiler_params=pltpu.CompilerParams(
            dimension_semantics=("parallel",),
        ),
    )(thr_arr, bn1r, bn2r, x0r, x1r)
    o0 = o0.reshape(B, H, W, C).transpose(0, 3, 1, 2)
    o1 = o1.reshape(B, H, W, C).transpose(0, 3, 1, 2)
    return o0, o1


def kernel(x0, x1, bn1_weight, bn2_weight, bn_threshold):
    return _run(x0, x1, bn1_weight, bn2_weight, bn_threshold)


# speedup vs baseline: 1.9734x; 1.0169x over previous
import jax
import jax.numpy as jnp
from jax.experimental import pallas as pl
from jax.experimental.pallas import tpu as pltpu

B, C, H, W = 8, 384, 56, 56
P1 = C // 2
N = B * H * W
RB = 3584
GRID = N // RB


def _body(thr_ref, bn1_ref, bn2_ref, x0_ref, x1_ref, o0_ref, o1_ref):
    thr = thr_ref[0, 0]
    c_idx = jax.lax.broadcasted_iota(jnp.int32, (1, C), 1)
    first = c_idx < P1
    bn1 = jnp.abs(bn1_ref[...])
    bn2 = jnp.abs(bn2_ref[...])
    keep0 = jnp.logical_or(first, bn1 > thr)
    take0 = jnp.logical_and(jnp.logical_not(first), bn1 < thr)
    keep1 = jnp.logical_or(first, bn2 > thr)
    take1 = jnp.logical_and(jnp.logical_not(first), bn2 < thr)
    x0 = x0_ref[...]
    x1 = x1_ref[...]
    zero = jnp.zeros_like(x0)
    o0_ref[...] = jnp.where(keep0, x0, jnp.where(take0, x1, zero))
    o1_ref[...] = jnp.where(keep1, x1, jnp.where(take1, x0, zero))


@jax.jit
def _run(x0, x1, bn1, bn2, thr):
    x0r = x0.transpose(0, 2, 3, 1).reshape(N, C)
    x1r = x1.transpose(0, 2, 3, 1).reshape(N, C)
    bn1r = bn1.reshape(1, C)
    bn2r = bn2.reshape(1, C)
    thr_arr = jnp.asarray(thr, jnp.float32).reshape(1, 1)
    data_spec = pl.BlockSpec((RB, C), lambda i: (i, 0))
    vec_spec = pl.BlockSpec((1, C), lambda i: (0, 0))
    thr_spec = pl.BlockSpec((1, 1), lambda i: (0, 0))
    o0, o1 = pl.pallas_call(
        _body,
        grid=(GRID,),
        in_specs=[thr_spec, vec_spec, vec_spec, data_spec, data_spec],
        out_specs=[data_spec, data_spec],
        out_shape=[
            jax.ShapeDtypeStruct((N, C), jnp.float32),
            jax.ShapeDtypeStruct((N, C), jnp.float32),
        ],
        compiler_params=pltpu.CompilerParams(
            dimension_semantics=("parallel",),
        ),
    )(thr_arr, bn1r, bn2r, x0r, x1r)
    o0 = o0.reshape(B, H, W, C).transpose(0, 3, 1, 2)
    o1 = o1.reshape(B, H, W, C).transpose(0, 3, 1, 2)
    return o0, o1


def kernel(x0, x1, bn1_weight, bn2_weight, bn_threshold):
    return _run(x0, x1, bn1_weight, bn2_weight, bn_threshold)
